# speedup vs baseline: 1.8313x; 1.8313x over previous
import functools

import jax
import jax.numpy as jnp
from jax import lax
from jax.experimental import pallas as pl
from jax.experimental.pallas import tpu as pltpu
from jax.experimental.pallas import tpu_sc as plsc

VOCAB = 8192
D = 8192
NIDX = 8192
NW = 32
BPW = NIDX // NW
K = 8
NCH = BPW // K

_mesh = plsc.VectorSubcoreMesh(core_axis_name="c", subcore_axis_name="s")


@functools.partial(
    pl.kernel,
    out_type=jax.ShapeDtypeStruct((NIDX, D), jnp.float32),
    mesh=_mesh,
    scratch_types=[
        pltpu.VMEM((BPW,), jnp.int32),
        pltpu.VMEM((K, D), jnp.float32),
        pltpu.SemaphoreType.DMA,
    ],
)
def _gather_rows(x_hbm, table_hbm, out_hbm, idx_v, rows_v, sem):
    wid = lax.axis_index("s") * 2 + lax.axis_index("c")
    base = wid * BPW
    pltpu.sync_copy(x_hbm.at[pl.ds(base, BPW)], idx_v)

    def body(c, carry):
        pltpu.async_copy(
            table_hbm.at[idx_v.at[pl.ds(c * K, K)]], rows_v, sem
        ).wait()
        pltpu.sync_copy(rows_v, out_hbm.at[pl.ds(base + c * K, K)])
        return carry

    lax.fori_loop(0, NCH, body, 0)


def kernel(x, table):
    out = _gather_rows(x.reshape(NIDX), table)
    return out.reshape(x.shape[0], x.shape[1], VOCAB)


# speedup vs baseline: 1.8906x; 1.0324x over previous
import functools

import jax
import jax.numpy as jnp
from jax import lax
from jax.experimental import pallas as pl
from jax.experimental.pallas import tpu as pltpu
from jax.experimental.pallas import tpu_sc as plsc

VOCAB = 8192
D = 8192
NIDX = 8192
NW = 32
BPW = NIDX // NW
K = 4
NCH = BPW // K

_mesh = plsc.VectorSubcoreMesh(core_axis_name="c", subcore_axis_name="s")


@functools.partial(
    pl.kernel,
    out_type=jax.ShapeDtypeStruct((NIDX, D), jnp.float32),
    mesh=_mesh,
    scratch_types=[
        pltpu.VMEM((NCH, K), jnp.int32),
        pltpu.VMEM((K, D), jnp.float32),
        pltpu.VMEM((K, D), jnp.float32),
        pltpu.SemaphoreType.DMA,
        pltpu.SemaphoreType.DMA,
        pltpu.SemaphoreType.DMA,
        pltpu.SemaphoreType.DMA,
    ],
)
def _gather_rows(x_hbm, table_hbm, out_hbm, idx_v, buf_a, buf_b,
                 gsem_a, gsem_b, ssem_a, ssem_b):
    wid = lax.axis_index("s") * 2 + lax.axis_index("c")
    base = wid * BPW
    pltpu.sync_copy(x_hbm.at[wid], idx_v)

    def gather(c, buf, sem):
        return pltpu.async_copy(table_hbm.at[idx_v.at[c]], buf, sem)

    def scatter(c, buf, sem):
        return pltpu.async_copy(buf, out_hbm.at[pl.ds(base + c * K, K)], sem)

    gather(0, buf_a, gsem_a)
    gather(1, buf_b, gsem_b)

    def body(i, carry):
        ca = 2 * i
        cb = 2 * i + 1
        pltpu.make_async_copy(table_hbm.at[idx_v.at[ca]], buf_a, gsem_a).wait()
        scatter(ca, buf_a, ssem_a)
        pltpu.make_async_copy(table_hbm.at[idx_v.at[cb]], buf_b, gsem_b).wait()
        scatter(cb, buf_b, ssem_b)
        pltpu.make_async_copy(
            buf_a, out_hbm.at[pl.ds(base + ca * K, K)], ssem_a).wait()
        gather(ca + 2, buf_a, gsem_a)
        pltpu.make_async_copy(
            buf_b, out_hbm.at[pl.ds(base + cb * K, K)], ssem_b).wait()
        gather(cb + 2, buf_b, gsem_b)
        return carry

    lax.fori_loop(0, NCH // 2 - 1, body, 0)

    ca = NCH - 2
    cb = NCH - 1
    pltpu.make_async_copy(table_hbm.at[idx_v.at[ca]], buf_a, gsem_a).wait()
    scatter(ca, buf_a, ssem_a)
    pltpu.make_async_copy(table_hbm.at[idx_v.at[cb]], buf_b, gsem_b).wait()
    scatter(cb, buf_b, ssem_b)
    pltpu.make_async_copy(
        buf_a, out_hbm.at[pl.ds(base + ca * K, K)], ssem_a).wait()
    pltpu.make_async_copy(
        buf_b, out_hbm.at[pl.ds(base + cb * K, K)], ssem_b).wait()


def kernel(x, table):
    out = _gather_rows(x.reshape(NW, NCH, K), table)
    return out.reshape(x.shape[0], x.shape[1], VOCAB)


# speedup vs baseline: 1.9596x; 1.0365x over previous
import functools

import jax
import jax.numpy as jnp
from jax import lax
from jax.experimental import pallas as pl
from jax.experimental.pallas import tpu as pltpu
from jax.experimental.pallas import tpu_sc as plsc

VOCAB = 8192
D = 8192
NIDX = 8192
NW = 32
BPW = NIDX // NW
K = 2
NCH = BPW // K
NBUF = 4
assert NCH % NBUF == 0
NOUT = NCH // NBUF
NTAIL = NBUF

_mesh = plsc.VectorSubcoreMesh(core_axis_name="c", subcore_axis_name="s")


@functools.partial(
    pl.kernel,
    out_type=jax.ShapeDtypeStruct((NIDX, D), jnp.float32),
    mesh=_mesh,
    scratch_types=(
        [pltpu.VMEM((NCH, K), jnp.int32)]
        + [pltpu.VMEM((K, D), jnp.float32) for _ in range(NBUF)]
        + [pltpu.SemaphoreType.DMA for _ in range(2 * NBUF)]
    ),
)
def _gather_rows(x_hbm, table_hbm, out_hbm, idx_v, *scratch):
    bufs = scratch[:NBUF]
    gsems = scratch[NBUF:2 * NBUF]
    ssems = scratch[2 * NBUF:3 * NBUF]
    wid = lax.axis_index("s") * 2 + lax.axis_index("c")
    base = wid * BPW
    pltpu.sync_copy(x_hbm.at[wid], idx_v)

    def gather(c, b):
        return pltpu.async_copy(table_hbm.at[idx_v.at[c]], bufs[b], gsems[b])

    def wait_gather(c, b):
        pltpu.make_async_copy(
            table_hbm.at[idx_v.at[c]], bufs[b], gsems[b]).wait()

    def scatter(c, b):
        return pltpu.async_copy(
            bufs[b], out_hbm.at[pl.ds(base + c * K, K)], ssems[b])

    def wait_scatter(c, b):
        pltpu.make_async_copy(
            bufs[b], out_hbm.at[pl.ds(base + c * K, K)], ssems[b]).wait()

    for b in range(NBUF):
        gather(b, b)

    def body(i, carry):
        c0 = i * NBUF
        for b in range(NBUF):
            c = c0 + b
            wait_gather(c, b)
            scatter(c, b)
            wait_scatter(c, b)
            gather(c + NBUF, b)
        return carry

    lax.fori_loop(0, NOUT - 1, body, 0)

    c0 = (NOUT - 1) * NBUF
    for b in range(NTAIL):
        c = c0 + b
        wait_gather(c, b)
        scatter(c, b)
    for b in range(NTAIL):
        wait_scatter(c0 + b, b)


def kernel(x, table):
    out = _gather_rows(x.reshape(NW, NCH, K), table)
    return out.reshape(x.shape[0], x.shape[1], VOCAB)
